# baseline (speedup 1.0000x reference)
import functools

import jax
import jax.numpy as jnp
from jax import lax
from jax.experimental import pallas as pl
from jax.experimental.pallas import tpu as pltpu
from jax.experimental.pallas import tpu_sc as plsc

N_USERS = 25000
N_ITEMS = 25000
DIM = 64
N_LAYERS = 3
N_EDGES = 800000
N_TOTAL = N_USERS + N_ITEMS

L = 16
NS = 16
NC = 2
NW = NC * NS

NP = 50176
NODES_W = NP // NW
PER_CORE = NP // NC
ACC_ROWS = 25600
DUMMY_ACC = PER_CORE
DUMMY_CNT = NP - 1

EP = 802816
EDGES_W = EP // NW
EDGES_T = EP // NS
CHUNK = 128

_mesh = plsc.VectorSubcoreMesh(core_axis_name="c", subcore_axis_name="s")


def _vrsqrt(x):
    i = lax.bitcast_convert_type(x, jnp.int32)
    i = jnp.int32(0x5F3759DF) - lax.shift_right_arithmetic(i, 1)
    y = lax.bitcast_convert_type(i, jnp.float32)
    for _ in range(3):
        y = y * (jnp.float32(1.5) - jnp.float32(0.5) * x * y * y)
    return y


K1CH = EDGES_W // CHUNK
K1SB = 14
K1NQ = K1CH // (2 * K1SB)

@functools.partial(
    pl.kernel,
    out_type=(
        jax.ShapeDtypeStruct((NC * NP,), jnp.float32),
        jax.ShapeDtypeStruct((NC * NP,), jnp.float32),
    ),
    mesh=_mesh,
    compiler_params=pltpu.CompilerParams(use_tc_tiling_on_sc=False),
    scratch_types=[
        pltpu.VMEM_SHARED((NP,), jnp.float32),
        pltpu.VMEM_SHARED((NP,), jnp.float32),
        pltpu.VMEM((3136,), jnp.float32),
        pltpu.VMEM((CHUNK,), jnp.float32),
        pltpu.VMEM((K1SB, CHUNK), jnp.int32),
        pltpu.VMEM((K1SB, CHUNK), jnp.int32),
        pltpu.VMEM((K1SB, CHUNK), jnp.int32),
        pltpu.VMEM((K1SB, CHUNK), jnp.int32),
        pltpu.SemaphoreType.DMA,
        pltpu.SemaphoreType.DMA,
        pltpu.SemaphoreType.DMA,
        pltpu.SemaphoreType.DMA,
        pltpu.SemaphoreType.DMA,
        pltpu.SemaphoreType.DMA,
    ],
)
def _k1_degrees(rows2, cols2, out_r, out_c, cnt_r, cnt_c, zbuf, ones,
                rb0, rb1, cb0, cb1, rs0, rs1, cs0, cs1, ts0, ts1):
    c = lax.axis_index("c")
    s = lax.axis_index("s")
    w = c * NS + s

    rib = (rb0, rb1)
    cib = (cb0, cb1)
    rsem = (rs0, rs1)
    csem = (cs0, cs1)
    tsem = (ts0, ts1)

    for t in range(3136 // L):
        zbuf[pl.ds(t * L, L)] = jnp.zeros((L,), jnp.float32)
    for t in range(CHUNK // L):
        ones[pl.ds(t * L, L)] = jnp.ones((L,), jnp.float32)
    pltpu.sync_copy(zbuf, cnt_r.at[pl.ds(s * 3136, 3136)])
    pltpu.sync_copy(zbuf, cnt_c.at[pl.ds(s * 3136, 3136)])

    def stage_start(blk, p):
        off = pl.multiple_of(w * K1CH + blk * K1SB, K1SB)
        pltpu.async_copy(rows2.at[pl.ds(off, K1SB)], rib[p], tsem[p])
        pltpu.async_copy(cols2.at[pl.ds(off, K1SB)], cib[p], tsem[p])

    def stage_wait(blk, p):
        off = pl.multiple_of(w * K1CH + blk * K1SB, K1SB)
        pltpu.make_async_copy(rows2.at[pl.ds(off, K1SB)], rib[p], tsem[p]).wait()
        pltpu.make_async_copy(cols2.at[pl.ds(off, K1SB)], cib[p], tsem[p]).wait()

    def xform(p):
        for i in range(K1SB):
            rr = rib[p].at[i]
            cr = cib[p].at[i]
            for t in range(CHUNK // L):
                sl = pl.ds(t * L, L)
                rv = rr[sl]
                rr[sl] = jnp.where(rv >= 0, rv, jnp.int32(DUMMY_CNT))
                cv = cr[sl]
                cr[sl] = jnp.where(cv >= 0, cv, jnp.int32(DUMMY_CNT))

    def sstart(p, i):
        pltpu.async_copy(ones, cnt_r.at[rib[p].at[i]], rsem[i % 2], add=True)
        pltpu.async_copy(ones, cnt_c.at[cib[p].at[i]], csem[i % 2], add=True)

    def swait(p, i):
        pltpu.make_async_copy(ones, cnt_r.at[rib[p].at[i]], rsem[i % 2]).wait()
        pltpu.make_async_copy(ones, cnt_c.at[cib[p].at[i]], csem[i % 2]).wait()

    def run_block(p):
        for i in range(K1SB):
            sstart(p, i)
            if i >= 2:
                swait(p, i - 2)

    def drain(p):
        for i in range(K1SB - 2, K1SB):
            swait(p, i)

    stage_start(0, 0)
    stage_start(1, 1)
    plsc.subcore_barrier()

    def pair(q, _):
        blk = 2 * q
        stage_wait(blk, 0)
        xform(0)
        stage_wait(blk + 1, 1)
        xform(1)
        run_block(0)
        drain(0)

        @pl.when(q < K1NQ - 1)
        def _():
            stage_start(blk + 2, 0)
        run_block(1)
        drain(1)

        @pl.when(q < K1NQ - 1)
        def _():
            stage_start(blk + 3, 1)
        return 0
    lax.fori_loop(0, K1NQ, pair, 0)

    plsc.subcore_barrier()
    evoff = pl.multiple_of(c * NP + s * 3136, CHUNK)
    pltpu.sync_copy(cnt_r.at[pl.ds(s * 3136, 3136)], zbuf)
    pltpu.sync_copy(zbuf, out_r.at[pl.ds(evoff, 3136)])
    pltpu.sync_copy(cnt_c.at[pl.ds(s * 3136, 3136)], zbuf)
    pltpu.sync_copy(zbuf, out_c.at[pl.ds(evoff, 3136)])


_K2R = 56
_K2C = NODES_W // _K2R
HD = DIM // 2

@functools.partial(
    pl.kernel,
    out_type=(
        jax.ShapeDtypeStruct((2 * NP, HD), jnp.float32),
        jax.ShapeDtypeStruct((NP, HD), jnp.float32),
        jax.ShapeDtypeStruct((NP, HD), jnp.float32),
    ),
    mesh=_mesh,
    compiler_params=pltpu.CompilerParams(use_tc_tiling_on_sc=False),
    scratch_types=[
        pltpu.VMEM((1600,), jnp.float32),
        pltpu.VMEM((1600,), jnp.float32),
        pltpu.VMEM((1600,), jnp.float32),
        pltpu.VMEM((1600,), jnp.float32),
        pltpu.VMEM((_K2R, DIM), jnp.float32),
        pltpu.VMEM((_K2R, DIM), jnp.float32),
        pltpu.VMEM((_K2R, HD), jnp.float32),
        pltpu.VMEM((_K2R, HD), jnp.float32),
        pltpu.VMEM((_K2R, HD), jnp.float32),
        pltpu.VMEM((_K2R, HD), jnp.float32),
        pltpu.SemaphoreType.DMA,
        pltpu.SemaphoreType.DMA,
    ],
)
def _k2_norm(pr, pc_, emb, emb0_o, smid_o, drow_o,
             ba, bb, drow, dcol, eb0, eb1, elo, ehi, sbuf, rbuf, is0, is1):
    c = lax.axis_index("c")
    s = lax.axis_index("s")
    base = pl.multiple_of((c * NS + s) * NODES_W, NODES_W)

    ebuf = (eb0, eb1)
    isem = (is0, is1)

    for part, dst in ((pr, drow), (pc_, dcol)):
        pltpu.sync_copy(part.at[pl.ds(base, NODES_W)],
                        ba.at[pl.ds(0, NODES_W)])
        pltpu.sync_copy(part.at[pl.ds(NP + base, NODES_W)],
                        bb.at[pl.ds(0, NODES_W)])
        for t in range(NODES_W // L):
            deg = ba[pl.ds(t * L, L)] + bb[pl.ds(t * L, L)]
            dst[pl.ds(t * L, L)] = _vrsqrt(jnp.maximum(deg, 1.0))

    def in_start(k, b):
        row0 = pl.multiple_of(base + k * _K2R, 8)
        pltpu.async_copy(emb.at[pl.ds(row0, _K2R)], ebuf[b], isem[b])

    def in_wait(k, b):
        row0 = pl.multiple_of(base + k * _K2R, 8)
        pltpu.make_async_copy(emb.at[pl.ds(row0, _K2R)], ebuf[b], isem[b]).wait()

    in_start(0, 0)

    def chunk2(q, _):
        for b in range(2):
            k = 2 * q + b

            @pl.when(k < _K2C - 1)
            def _():
                in_start(k + 1, 1 - b)
            in_wait(k, b)
            eb = ebuf[b]
            for g in range(_K2R // L + 1):
                if g * L >= _K2R:
                    break
                v_dc = dcol[pl.ds(k * _K2R + g * L, L)]
                v_dr = drow[pl.ds(k * _K2R + g * L, L)]
                v_dm = v_dc * v_dr
                for rr in range(min(L, _K2R - g * L)):
                    r = g * L + rr
                    bc = jnp.broadcast_to(v_dc[rr], (L,))
                    bm = jnp.broadcast_to(v_dm[rr], (L,))
                    br = jnp.broadcast_to(v_dr[rr], (L,))
                    for t in range(HD // L):
                        sl = pl.ds(t * L, L)
                        elo[r, sl] = eb[r, sl] * bc
                        ehi[r, sl] = eb[r, pl.ds(HD + t * L, L)] * bc
                        sbuf[r, sl] = bm
                        rbuf[r, sl] = br
            row0 = pl.multiple_of(base + k * _K2R, 8)
            pltpu.sync_copy(elo, emb0_o.at[pl.ds(row0, _K2R)])
            pltpu.sync_copy(ehi, emb0_o.at[pl.ds(NP + row0, _K2R)])
            pltpu.sync_copy(sbuf, smid_o.at[pl.ds(row0, _K2R)])
            pltpu.sync_copy(rbuf, drow_o.at[pl.ds(row0, _K2R)])
        return 0
    lax.fori_loop(0, _K2C // 2, chunk2, 0)


ACC2 = 51200
DUMMY2 = NP
NCH = EDGES_T // CHUNK
SB = 14
NQ = NCH // (2 * SB)
NMS = 4

@functools.partial(
    pl.kernel,
    out_type=jax.ShapeDtypeStruct((4 * NP, HD), jnp.float32),
    mesh=_mesh,
    compiler_params=pltpu.CompilerParams(use_tc_tiling_on_sc=False),
    scratch_types=[
        pltpu.VMEM_SHARED((ACC2, HD), jnp.float32),
        pltpu.VMEM((CHUNK, HD), jnp.float32),
        pltpu.VMEM((CHUNK, HD), jnp.float32),
        pltpu.VMEM((CHUNK, HD), jnp.float32),
        pltpu.VMEM((CHUNK, HD), jnp.float32),
        pltpu.VMEM((SB, CHUNK), jnp.int32),
        pltpu.VMEM((SB, CHUNK), jnp.int32),
        pltpu.VMEM((SB, CHUNK), jnp.int32),
        pltpu.VMEM((SB, CHUNK), jnp.int32),
        pltpu.SemaphoreType.DMA,
        pltpu.SemaphoreType.DMA,
        pltpu.SemaphoreType.DMA,
        pltpu.SemaphoreType.DMA,
        pltpu.SemaphoreType.DMA,
        pltpu.SemaphoreType.DMA,
        pltpu.SemaphoreType.DMA,
        pltpu.SemaphoreType.DMA,
        pltpu.SemaphoreType.DMA,
        pltpu.SemaphoreType.DMA,
    ],
)
def _k3_layers(emb0, rows2, cols2, scale2, oo,
               acc, msg0, msg1, msg2, msg3, sb0, sb1, gb0, gb1,
               gsem0, gsem1, gsem2, gsem3, ssem0, ssem1, ssem2, ssem3,
               tsem0, tsem1):
    c = lax.axis_index("c")
    s = lax.axis_index("s")
    tbase = c * NP

    msg = (msg0, msg1, msg2, msg3)
    sib = (sb0, sb1)
    gib = (gb0, gb1)
    gsem = (gsem0, gsem1, gsem2, gsem3)
    ssem = (ssem0, ssem1, ssem2, ssem3)
    tsem = (tsem0, tsem1)

    hold = {}

    def stage_start(blk, p):
        off = pl.multiple_of(s * NCH + blk * SB, SB)
        pltpu.async_copy(rows2.at[pl.ds(off, SB)], sib[p], tsem[p])
        pltpu.async_copy(cols2.at[pl.ds(off, SB)], gib[p], tsem[p])

    def stage_wait(blk, p):
        off = pl.multiple_of(s * NCH + blk * SB, SB)
        pltpu.make_async_copy(rows2.at[pl.ds(off, SB)], sib[p], tsem[p]).wait()
        pltpu.make_async_copy(cols2.at[pl.ds(off, SB)], gib[p], tsem[p]).wait()

    def xform(p):
        for i in range(SB):
            gr = gib[p].at[i]
            sr = sib[p].at[i]
            for t in range(CHUNK // L):
                sl = pl.ds(t * L, L)
                gr[sl] = jnp.maximum(gr[sl], 0) + hold['gbase']
                rv = sr[sl]
                sr[sl] = jnp.where(rv >= 0, rv, jnp.int32(DUMMY2))

    def gstart(p, i, b):
        pltpu.async_copy(oo.at[gib[p].at[i]], msg[b], gsem[b])

    def gwait(p, i, b):
        pltpu.make_async_copy(oo.at[gib[p].at[i]], msg[b], gsem[b]).wait()

    def sstart(p, i, b):
        pltpu.async_copy(msg[b], acc.at[sib[p].at[i]], ssem[b], add=True)

    def swait(p, i, b):
        pltpu.make_async_copy(msg[b], acc.at[sib[p].at[i]], ssem[b]).wait()

    def run_block(p, off):
        for i in range(SB):
            st = (i + off) % NMS
            gwait(p, i, st)
            sstart(p, i, st)
            if i >= NMS - 1:
                swait(p, i - (NMS - 1), (i - (NMS - 1) + off) % NMS)
            if i < SB - 1:
                gstart(p, i + 1, (i + 1 + off) % NMS)

    def drain(p, off):
        for i in range(SB - (NMS - 1), SB):
            swait(p, i, (i + off) % NMS)

    def seed_chunk(start, nrows):
        pltpu.sync_copy(emb0.at[pl.ds(tbase + start, nrows)],
                        msg0.at[pl.ds(0, nrows)] if nrows != CHUNK else msg0)
        pltpu.sync_copy(msg0.at[pl.ds(0, nrows)] if nrows != CHUNK else msg0,
                        oo.at[pl.ds(tbase + start, nrows)])

    def seed(k, _):
        start = pl.multiple_of(s * (NP // NS) + k * CHUNK, 8)
        seed_chunk(start, CHUNK)
        return 0
    lax.fori_loop(0, (NP // NS) // CHUNK, seed, 0)
    stail = pl.multiple_of(s * (NP // NS) + ((NP // NS) // CHUNK) * CHUNK, 8)
    seed_chunk(stail, (NP // NS) % CHUNK)

    def one_layer(l, _):
        in_off = pl.multiple_of(lax.rem(l, 2) * (2 * NP), 8)
        out_off = pl.multiple_of((1 - lax.rem(l, 2)) * (2 * NP), 8)
        scl_off = pl.multiple_of(jnp.where(l == 2, NP, 0), 8)
        hold['gbase'] = in_off + tbase

        for r in range(CHUNK):
            for t in range(HD // L):
                msg0[r, pl.ds(t * L, L)] = jnp.zeros((L,), jnp.float32)
        for k in range(ACC2 // NS // CHUNK):
            pltpu.sync_copy(msg0,
                            acc.at[pl.ds(s * (ACC2 // NS) + k * CHUNK, CHUNK)])
        stage_start(0, 0)
        stage_start(1, 1)
        plsc.subcore_barrier()

        def pair(q, _):
            blk = 2 * q
            stage_wait(blk, 0)
            xform(0)
            gstart(0, 0, 0)
            stage_wait(blk + 1, 1)
            xform(1)
            run_block(0, 0)
            drain(0, 0)

            @pl.when(q < NQ - 1)
            def _():
                stage_start(blk + 2, 0)
            gstart(1, 0, SB % NMS)
            run_block(1, SB % NMS)
            drain(1, SB % NMS)

            @pl.when(q < NQ - 1)
            def _():
                stage_start(blk + 3, 1)
            return 0
        lax.fori_loop(0, NQ, pair, 0)

        plsc.subcore_barrier()

        def evac_chunk(start, nrows):
            pltpu.sync_copy(acc.at[pl.ds(start, nrows)],
                            msg0.at[pl.ds(0, nrows)] if nrows != CHUNK else msg0)
            pltpu.sync_copy(scale2.at[pl.ds(scl_off + start, nrows)],
                            msg1.at[pl.ds(0, nrows)] if nrows != CHUNK else msg1)
            for r in range(nrows):
                for t in range(HD // L):
                    sl = pl.ds(t * L, L)
                    msg0[r, sl] = msg0[r, sl] * msg1[r, sl]
            pltpu.sync_copy(msg0.at[pl.ds(0, nrows)] if nrows != CHUNK else msg0,
                            oo.at[pl.ds(out_off + tbase + start, nrows)])

        def evac(k, _):
            start = pl.multiple_of(s * (NP // NS) + k * CHUNK, 8)
            evac_chunk(start, CHUNK)
            return 0
        lax.fori_loop(0, (NP // NS) // CHUNK, evac, 0)
        tail = pl.multiple_of(s * (NP // NS) + ((NP // NS) // CHUNK) * CHUNK, 8)
        evac_chunk(tail, (NP // NS) % CHUNK)
        plsc.subcore_barrier()
        return 0
    lax.fori_loop(0, 3, one_layer, 0)


def kernel(edge_index, user_weight, item_weight):
    edges = jnp.pad(edge_index, ((0, 0), (0, EP - N_EDGES)),
                    constant_values=-1)
    rows = edges[0]
    cols = edges[1]
    rows2x = rows.reshape(EP // CHUNK, CHUNK)
    cols2x = cols.reshape(EP // CHUNK, CHUNK)
    all_emb = jnp.concatenate([user_weight, item_weight], axis=0)
    all_emb = jnp.pad(all_emb, ((0, NP - N_TOTAL), (0, 0)))

    part_r, part_c = _k1_degrees(rows2x, cols2x)
    emb0, smid, drow = _k2_norm(part_r, part_c, all_emb)

    scale2 = jnp.concatenate([smid, drow], axis=0)
    oo = _k3_layers(emb0, rows2x, cols2x, scale2)
    e = oo[2 * NP:]

    full = jnp.concatenate([e[:N_TOTAL], e[NP:NP + N_TOTAL]], axis=1)
    return (full[:N_USERS], full[N_USERS:N_TOTAL])


# speedup vs baseline: 1.1154x; 1.1154x over previous
import functools

import jax
import jax.numpy as jnp
from jax import lax
from jax.experimental import pallas as pl
from jax.experimental.pallas import tpu as pltpu
from jax.experimental.pallas import tpu_sc as plsc

N_USERS = 25000
N_ITEMS = 25000
DIM = 64
N_LAYERS = 3
N_EDGES = 800000
N_TOTAL = N_USERS + N_ITEMS

L = 16
NS = 16
NC = 2
NW = NC * NS

NP = 50176
NODES_W = NP // NW
PER_CORE = NP // NC
ACC_ROWS = 25600
DUMMY_ACC = PER_CORE
DUMMY_CNT = NP - 1

EP = 802816
EDGES_W = EP // NW
EDGES_T = EP // NS
CHUNK = 128

_mesh = plsc.VectorSubcoreMesh(core_axis_name="c", subcore_axis_name="s")


def _vrsqrt(x):
    i = lax.bitcast_convert_type(x, jnp.int32)
    i = jnp.int32(0x5F3759DF) - lax.shift_right_arithmetic(i, 1)
    y = lax.bitcast_convert_type(i, jnp.float32)
    for _ in range(3):
        y = y * (jnp.float32(1.5) - jnp.float32(0.5) * x * y * y)
    return y


K1CH = EDGES_W // CHUNK
K1SB = 14
K1NQ = K1CH // (2 * K1SB)

@functools.partial(
    pl.kernel,
    out_type=(
        jax.ShapeDtypeStruct((NC * NP,), jnp.float32),
        jax.ShapeDtypeStruct((NC * NP,), jnp.float32),
    ),
    mesh=_mesh,
    compiler_params=pltpu.CompilerParams(use_tc_tiling_on_sc=False),
    scratch_types=[
        pltpu.VMEM_SHARED((NP,), jnp.float32),
        pltpu.VMEM_SHARED((NP,), jnp.float32),
        pltpu.VMEM((3136,), jnp.float32),
        pltpu.VMEM((CHUNK,), jnp.float32),
        pltpu.VMEM((K1SB, CHUNK), jnp.int32),
        pltpu.VMEM((K1SB, CHUNK), jnp.int32),
        pltpu.VMEM((K1SB, CHUNK), jnp.int32),
        pltpu.VMEM((K1SB, CHUNK), jnp.int32),
        pltpu.SemaphoreType.DMA,
        pltpu.SemaphoreType.DMA,
        pltpu.SemaphoreType.DMA,
        pltpu.SemaphoreType.DMA,
        pltpu.SemaphoreType.DMA,
        pltpu.SemaphoreType.DMA,
    ],
)
def _k1_degrees(rows2, cols2, out_r, out_c, cnt_r, cnt_c, zbuf, ones,
                rb0, rb1, cb0, cb1, rs0, rs1, cs0, cs1, ts0, ts1):
    c = lax.axis_index("c")
    s = lax.axis_index("s")
    w = c * NS + s

    rib = (rb0, rb1)
    cib = (cb0, cb1)
    rsem = (rs0, rs1)
    csem = (cs0, cs1)
    tsem = (ts0, ts1)

    for t in range(3136 // L):
        zbuf[pl.ds(t * L, L)] = jnp.zeros((L,), jnp.float32)
    for t in range(CHUNK // L):
        ones[pl.ds(t * L, L)] = jnp.ones((L,), jnp.float32)
    pltpu.sync_copy(zbuf, cnt_r.at[pl.ds(s * 3136, 3136)])
    pltpu.sync_copy(zbuf, cnt_c.at[pl.ds(s * 3136, 3136)])

    def stage_start(blk, p):
        off = pl.multiple_of(w * K1CH + blk * K1SB, K1SB)
        pltpu.async_copy(rows2.at[pl.ds(off, K1SB)], rib[p], tsem[p])
        pltpu.async_copy(cols2.at[pl.ds(off, K1SB)], cib[p], tsem[p])

    def stage_wait(blk, p):
        off = pl.multiple_of(w * K1CH + blk * K1SB, K1SB)
        pltpu.make_async_copy(rows2.at[pl.ds(off, K1SB)], rib[p], tsem[p]).wait()
        pltpu.make_async_copy(cols2.at[pl.ds(off, K1SB)], cib[p], tsem[p]).wait()

    def xform(p):
        for i in range(K1SB):
            rr = rib[p].at[i]
            cr = cib[p].at[i]
            for t in range(CHUNK // L):
                sl = pl.ds(t * L, L)
                rv = rr[sl]
                rr[sl] = jnp.where(rv >= 0, rv, jnp.int32(DUMMY_CNT))
                cv = cr[sl]
                cr[sl] = jnp.where(cv >= 0, cv, jnp.int32(DUMMY_CNT))

    def sstart(p, i):
        pltpu.async_copy(ones, cnt_r.at[rib[p].at[i]], rsem[i % 2], add=True)
        pltpu.async_copy(ones, cnt_c.at[cib[p].at[i]], csem[i % 2], add=True)

    def swait(p, i):
        pltpu.make_async_copy(ones, cnt_r.at[rib[p].at[i]], rsem[i % 2]).wait()
        pltpu.make_async_copy(ones, cnt_c.at[cib[p].at[i]], csem[i % 2]).wait()

    def run_block(p):
        for i in range(K1SB):
            sstart(p, i)
            if i >= 2:
                swait(p, i - 2)

    def drain(p):
        for i in range(K1SB - 2, K1SB):
            swait(p, i)

    stage_start(0, 0)
    stage_start(1, 1)
    plsc.subcore_barrier()

    def pair(q, _):
        blk = 2 * q
        stage_wait(blk, 0)
        xform(0)
        stage_wait(blk + 1, 1)
        xform(1)
        run_block(0)
        drain(0)

        @pl.when(q < K1NQ - 1)
        def _():
            stage_start(blk + 2, 0)
        run_block(1)
        drain(1)

        @pl.when(q < K1NQ - 1)
        def _():
            stage_start(blk + 3, 1)
        return 0
    lax.fori_loop(0, K1NQ, pair, 0)

    plsc.subcore_barrier()
    evoff = pl.multiple_of(c * NP + s * 3136, CHUNK)
    pltpu.sync_copy(cnt_r.at[pl.ds(s * 3136, 3136)], zbuf)
    pltpu.sync_copy(zbuf, out_r.at[pl.ds(evoff, 3136)])
    pltpu.sync_copy(cnt_c.at[pl.ds(s * 3136, 3136)], zbuf)
    pltpu.sync_copy(zbuf, out_c.at[pl.ds(evoff, 3136)])


_K2R = 56
_K2C = NODES_W // _K2R
HD = DIM // 2

@functools.partial(
    pl.kernel,
    out_type=(
        jax.ShapeDtypeStruct((2 * NP, HD), jnp.float32),
        jax.ShapeDtypeStruct((NP, HD), jnp.float32),
        jax.ShapeDtypeStruct((NP, HD), jnp.float32),
    ),
    mesh=_mesh,
    compiler_params=pltpu.CompilerParams(use_tc_tiling_on_sc=False),
    scratch_types=[
        pltpu.VMEM((1600,), jnp.float32),
        pltpu.VMEM((1600,), jnp.float32),
        pltpu.VMEM((1600,), jnp.float32),
        pltpu.VMEM((1600,), jnp.float32),
        pltpu.VMEM((_K2R, DIM), jnp.float32),
        pltpu.VMEM((_K2R, DIM), jnp.float32),
        pltpu.VMEM((_K2R, HD), jnp.float32),
        pltpu.VMEM((_K2R, HD), jnp.float32),
        pltpu.VMEM((_K2R, HD), jnp.float32),
        pltpu.VMEM((_K2R, HD), jnp.float32),
        pltpu.SemaphoreType.DMA,
        pltpu.SemaphoreType.DMA,
    ],
)
def _k2_norm(pr, pc_, emb, emb0_o, smid_o, drow_o,
             ba, bb, drow, dcol, eb0, eb1, elo, ehi, sbuf, rbuf, is0, is1):
    c = lax.axis_index("c")
    s = lax.axis_index("s")
    base = pl.multiple_of((c * NS + s) * NODES_W, NODES_W)

    ebuf = (eb0, eb1)
    isem = (is0, is1)

    for part, dst in ((pr, drow), (pc_, dcol)):
        pltpu.sync_copy(part.at[pl.ds(base, NODES_W)],
                        ba.at[pl.ds(0, NODES_W)])
        pltpu.sync_copy(part.at[pl.ds(NP + base, NODES_W)],
                        bb.at[pl.ds(0, NODES_W)])
        for t in range(NODES_W // L):
            deg = ba[pl.ds(t * L, L)] + bb[pl.ds(t * L, L)]
            dst[pl.ds(t * L, L)] = _vrsqrt(jnp.maximum(deg, 1.0))

    def in_start(k, b):
        row0 = pl.multiple_of(base + k * _K2R, 8)
        pltpu.async_copy(emb.at[pl.ds(row0, _K2R)], ebuf[b], isem[b])

    def in_wait(k, b):
        row0 = pl.multiple_of(base + k * _K2R, 8)
        pltpu.make_async_copy(emb.at[pl.ds(row0, _K2R)], ebuf[b], isem[b]).wait()

    in_start(0, 0)

    def chunk2(q, _):
        for b in range(2):
            k = 2 * q + b

            @pl.when(k < _K2C - 1)
            def _():
                in_start(k + 1, 1 - b)
            in_wait(k, b)
            eb = ebuf[b]
            for g in range(_K2R // L + 1):
                if g * L >= _K2R:
                    break
                v_dc = dcol[pl.ds(k * _K2R + g * L, L)]
                v_dr = drow[pl.ds(k * _K2R + g * L, L)]
                v_dm = v_dc * v_dr
                for rr in range(min(L, _K2R - g * L)):
                    r = g * L + rr
                    bc = jnp.broadcast_to(v_dc[rr], (L,))
                    bm = jnp.broadcast_to(v_dm[rr], (L,))
                    br = jnp.broadcast_to(v_dr[rr], (L,))
                    for t in range(HD // L):
                        sl = pl.ds(t * L, L)
                        elo[r, sl] = eb[r, sl] * bc
                        ehi[r, sl] = eb[r, pl.ds(HD + t * L, L)] * bc
                        sbuf[r, sl] = bm
                        rbuf[r, sl] = br
            row0 = pl.multiple_of(base + k * _K2R, 8)
            pltpu.sync_copy(elo, emb0_o.at[pl.ds(row0, _K2R)])
            pltpu.sync_copy(ehi, emb0_o.at[pl.ds(NP + row0, _K2R)])
            pltpu.sync_copy(sbuf, smid_o.at[pl.ds(row0, _K2R)])
            pltpu.sync_copy(rbuf, drow_o.at[pl.ds(row0, _K2R)])
        return 0
    lax.fori_loop(0, _K2C // 2, chunk2, 0)


ACC2 = 51200
DUMMY2 = NP
NCH = EDGES_T // CHUNK
SB = 14
NQ = NCH // (2 * SB)
NMS = 4

@functools.partial(
    pl.kernel,
    out_type=jax.ShapeDtypeStruct((2 * NP, HD), jnp.float32),
    mesh=_mesh,
    compiler_params=pltpu.CompilerParams(use_tc_tiling_on_sc=False),
    scratch_types=[
        pltpu.VMEM_SHARED((ACC2, HD), jnp.float32),
        pltpu.VMEM((CHUNK, HD), jnp.float32),
        pltpu.VMEM((CHUNK, HD), jnp.float32),
        pltpu.VMEM((CHUNK, HD), jnp.float32),
        pltpu.VMEM((CHUNK, HD), jnp.float32),
        pltpu.VMEM((SB, CHUNK), jnp.int32),
        pltpu.VMEM((SB, CHUNK), jnp.int32),
        pltpu.VMEM((SB, CHUNK), jnp.int32),
        pltpu.VMEM((SB, CHUNK), jnp.int32),
        pltpu.SemaphoreType.DMA,
        pltpu.SemaphoreType.DMA,
        pltpu.SemaphoreType.DMA,
        pltpu.SemaphoreType.DMA,
        pltpu.SemaphoreType.DMA,
        pltpu.SemaphoreType.DMA,
        pltpu.SemaphoreType.DMA,
        pltpu.SemaphoreType.DMA,
        pltpu.SemaphoreType.DMA,
        pltpu.SemaphoreType.DMA,
    ],
)
def _k3_layer(embs, rows2, cols2, scale, out,
              acc, msg0, msg1, msg2, msg3, sb0, sb1, gb0, gb1,
              gsem0, gsem1, gsem2, gsem3, ssem0, ssem1, ssem2, ssem3,
              tsem0, tsem1):
    c = lax.axis_index("c")
    s = lax.axis_index("s")
    tbase = c * NP

    msg = (msg0, msg1, msg2, msg3)
    sib = (sb0, sb1)
    gib = (gb0, gb1)
    gsem = (gsem0, gsem1, gsem2, gsem3)
    ssem = (ssem0, ssem1, ssem2, ssem3)
    tsem = (tsem0, tsem1)

    for r in range(CHUNK):
        for t in range(HD // L):
            msg0[r, pl.ds(t * L, L)] = jnp.zeros((L,), jnp.float32)
    for k in range(ACC2 // NS // CHUNK):
        pltpu.sync_copy(msg0, acc.at[pl.ds(s * (ACC2 // NS) + k * CHUNK, CHUNK)])

    def stage_start(blk, p):
        off = pl.multiple_of(s * NCH + blk * SB, SB)
        pltpu.async_copy(rows2.at[pl.ds(off, SB)], sib[p], tsem[p])
        pltpu.async_copy(cols2.at[pl.ds(off, SB)], gib[p], tsem[p])

    def stage_wait(blk, p):
        off = pl.multiple_of(s * NCH + blk * SB, SB)
        pltpu.make_async_copy(rows2.at[pl.ds(off, SB)], sib[p], tsem[p]).wait()
        pltpu.make_async_copy(cols2.at[pl.ds(off, SB)], gib[p], tsem[p]).wait()

    def xform(p):
        for i in range(SB):
            gr = gib[p].at[i]
            sr = sib[p].at[i]
            for t in range(CHUNK // L):
                sl = pl.ds(t * L, L)
                gr[sl] = jnp.maximum(gr[sl], 0) + tbase
                rv = sr[sl]
                sr[sl] = jnp.where(rv >= 0, rv, jnp.int32(DUMMY2))

    def gstart(p, i, b):
        pltpu.async_copy(embs.at[gib[p].at[i]], msg[b], gsem[b])

    def gwait(p, i, b):
        pltpu.make_async_copy(embs.at[gib[p].at[i]], msg[b], gsem[b]).wait()

    def sstart(p, i, b):
        pltpu.async_copy(msg[b], acc.at[sib[p].at[i]], ssem[b], add=True)

    def swait(p, i, b):
        pltpu.make_async_copy(msg[b], acc.at[sib[p].at[i]], ssem[b]).wait()

    def run_block(p, off):
        for i in range(SB):
            st = (i + off) % NMS
            gwait(p, i, st)
            sstart(p, i, st)
            if i >= NMS - 1:
                swait(p, i - (NMS - 1), (i - (NMS - 1) + off) % NMS)
            if i < SB - 1:
                gstart(p, i + 1, (i + 1 + off) % NMS)

    def drain(p, off):
        for i in range(SB - (NMS - 1), SB):
            swait(p, i, (i + off) % NMS)

    stage_start(0, 0)
    stage_start(1, 1)
    plsc.subcore_barrier()

    def pair(q, _):
        blk = 2 * q
        stage_wait(blk, 0)
        xform(0)
        gstart(0, 0, 0)
        stage_wait(blk + 1, 1)
        xform(1)
        run_block(0, 0)
        drain(0, 0)

        @pl.when(q < NQ - 1)
        def _():
            stage_start(blk + 2, 0)
        gstart(1, 0, SB % NMS)
        run_block(1, SB % NMS)
        drain(1, SB % NMS)

        @pl.when(q < NQ - 1)
        def _():
            stage_start(blk + 3, 1)
        return 0
    lax.fori_loop(0, NQ, pair, 0)

    plsc.subcore_barrier()

    def evac_chunk(start, nrows):
        pltpu.sync_copy(acc.at[pl.ds(start, nrows)],
                        msg0.at[pl.ds(0, nrows)] if nrows != CHUNK else msg0)
        pltpu.sync_copy(scale.at[pl.ds(start, nrows)],
                        msg1.at[pl.ds(0, nrows)] if nrows != CHUNK else msg1)
        for r in range(nrows):
            for t in range(HD // L):
                sl = pl.ds(t * L, L)
                msg0[r, sl] = msg0[r, sl] * msg1[r, sl]
        pltpu.sync_copy(msg0.at[pl.ds(0, nrows)] if nrows != CHUNK else msg0,
                        out.at[pl.ds(tbase + start, nrows)])

    def evac(k, _):
        start = pl.multiple_of(s * (NP // NS) + k * CHUNK, 8)
        evac_chunk(start, CHUNK)
        return 0
    lax.fori_loop(0, (NP // NS) // CHUNK, evac, 0)
    tail = pl.multiple_of(s * (NP // NS) + ((NP // NS) // CHUNK) * CHUNK, 8)
    evac_chunk(tail, (NP // NS) % CHUNK)


def kernel(edge_index, user_weight, item_weight):
    edges = jnp.pad(edge_index, ((0, 0), (0, EP - N_EDGES)),
                    constant_values=-1)
    rows = edges[0]
    cols = edges[1]
    rows2x = rows.reshape(EP // CHUNK, CHUNK)
    cols2x = cols.reshape(EP // CHUNK, CHUNK)
    all_emb = jnp.concatenate([user_weight, item_weight], axis=0)
    all_emb = jnp.pad(all_emb, ((0, NP - N_TOTAL), (0, 0)))

    part_r, part_c = _k1_degrees(rows2x, cols2x)
    emb0, smid, drow = _k2_norm(part_r, part_c, all_emb)

    e = _k3_layer(emb0, rows2x, cols2x, smid)
    e = _k3_layer(e, rows2x, cols2x, smid)
    e = _k3_layer(e, rows2x, cols2x, drow)

    full = jnp.concatenate([e[:N_TOTAL], e[NP:NP + N_TOTAL]], axis=1)
    return (full[:N_USERS], full[N_USERS:N_TOTAL])
